# baseline (speedup 1.0000x reference)
import jax
import jax.numpy as jnp
from jax.experimental import pallas as pl
from jax.experimental.pallas import tpu as pltpu

BM = 2048
BK = 512
N_ITEMS = 4096
K_STEPS = N_ITEMS // BK


def _matmul_zero_diag_kernel(x_ref, w_ref, o_hbm, acc_ref, sem):
    mi = pl.program_id(0)
    kk = pl.program_id(1)

    @pl.when(kk == 0)
    def _():
        acc_ref[...] = jnp.zeros_like(acc_ref)

    acc_ref[...] += jnp.dot(
        x_ref[...], w_ref[...], preferred_element_type=jnp.float32
    )

    wsub = w_ref[:, pl.ds(kk * BK, BK)]
    is_diag = (
        jax.lax.broadcasted_iota(jnp.int32, (BK, BK), 0)
        == jax.lax.broadcasted_iota(jnp.int32, (BK, BK), 1)
    )
    d = jnp.sum(jnp.where(is_diag, wsub, 0.0), axis=0, keepdims=True)
    acc_ref[:, pl.ds(kk * BK, BK)] -= x_ref[...] * d

    @pl.when(kk == K_STEPS - 1)
    def _():
        copy = pltpu.make_async_copy(
            acc_ref, o_hbm.at[pl.ds(mi * BM, BM), :], sem
        )
        copy.start()
        copy.wait()


@jax.jit
def kernel(x, W):
    M, K = x.shape
    _, N = W.shape
    grid = (M // BM, K // BK)
    return pl.pallas_call(
        _matmul_zero_diag_kernel,
        grid=grid,
        in_specs=[
            pl.BlockSpec((BM, BK), lambda mi, kk: (mi, kk)),
            pl.BlockSpec((BK, N), lambda mi, kk: (kk, 0)),
        ],
        out_specs=pl.BlockSpec(memory_space=pltpu.MemorySpace.HBM),
        out_shape=jax.ShapeDtypeStruct((M, N), jnp.float32),
        scratch_shapes=[
            pltpu.VMEM((BM, N), jnp.float32),
            pltpu.SemaphoreType.DMA,
        ],
        compiler_params=pltpu.CompilerParams(
            dimension_semantics=("parallel", "arbitrary"),
            vmem_limit_bytes=64 * 1024 * 1024,
        ),
    )(x, W)
